# Initial kernel scaffold; baseline (speedup 1.0000x reference)
import jax
import jax.numpy as jnp
from jax.experimental import pallas as pl


def kernel(x, logits, W0, ln_gamma, ln_beta):
    raise NotImplementedError("write your pallas kernel here")


# speedup vs baseline: 3.0051x; 3.0051x over previous
import functools

import jax
import jax.numpy as jnp
from jax.experimental import pallas as pl

_B, _S, _D, _V = 4, 2048, 2048, 8192
_LR = 1e-4
_CORRUPT = 0.8
_T0, _T1 = 0.9, 0.945
import math
_INV_LOGV = float(1.0 / math.log(float(_V)))

_TBLK = 256


def _fused_body(x_ref, logits_ref, w_ref, g_ref, b_ref, out_ref):
    l = logits_ref[...]
    m = jnp.max(l, axis=-1, keepdims=True)
    e = jnp.exp(l - m)
    se = jnp.sum(e, axis=-1, keepdims=True)
    sl = jnp.sum(e * l, axis=-1, keepdims=True)
    ent = m + jnp.log(se) - sl / se
    diff = ent * _INV_LOGV

    xv = x_ref[...]
    mu = jnp.mean(xv, axis=-1, keepdims=True)
    xc0 = xv - mu
    var = jnp.mean(xc0 * xc0, axis=-1, keepdims=True)
    xn = xc0 * jax.lax.rsqrt(var + 1e-5) * g_ref[...] + b_ref[...]
    x_c = xn * _CORRUPT
    s = _LR * jnp.sum(x_c * x_c, axis=-1, keepdims=True)

    one_minus_s = 1.0 - s
    c2 = one_minus_s * one_minus_s
    c4 = c2 * c2
    c = jnp.where(diff < _T0, one_minus_s, jnp.where(diff < _T1, c2, c4))

    p0 = jnp.dot(x_c.astype(jnp.bfloat16), w_ref[...],
                 preferred_element_type=jnp.float32)

    out_ref[...] = xv + xn + c * (p0 - xn)


@jax.jit
def kernel(x, logits, W0, ln_gamma, ln_beta):
    n_tok = _B * _S
    x2 = x.reshape(n_tok, _D)
    l2 = logits.reshape(n_tok, _V)
    g2 = ln_gamma.reshape(1, _D)
    b2 = ln_beta.reshape(1, _D)
    w_bf16 = W0.astype(jnp.bfloat16)

    grid = (n_tok // _TBLK,)
    out = pl.pallas_call(
        _fused_body,
        grid=grid,
        in_specs=[
            pl.BlockSpec((_TBLK, _D), lambda i: (i, 0)),
            pl.BlockSpec((_TBLK, _V), lambda i: (i, 0)),
            pl.BlockSpec((_D, _D), lambda i: (0, 0)),
            pl.BlockSpec((1, _D), lambda i: (0, 0)),
            pl.BlockSpec((1, _D), lambda i: (0, 0)),
        ],
        out_specs=pl.BlockSpec((_TBLK, _D), lambda i: (i, 0)),
        out_shape=jax.ShapeDtypeStruct((n_tok, _D), jnp.float32),
    )(x2, l2, w_bf16, g2, b2)
    return out.reshape(_B, _S, _D)
